# baseline (speedup 1.0000x reference)
import functools
import math

import numpy as np
import jax
import jax.numpy as jnp
from jax import lax
from jax.experimental import pallas as pl
from jax.experimental.pallas import tpu as pltpu
from jax.experimental.pallas import tpu_sc as plsc

_MAX_SEQ = 200
_D = 64
_L = 16
_NC, _NS = 2, 16
_NW = _NC * _NS


def _pe_np(max_seq, d):
    pos = np.arange(max_seq, dtype=np.float32)[:, None]
    even_idx = np.arange(0, d, 2, dtype=np.float32)
    odd_idx = np.arange(1, d, 2, dtype=np.float32)
    even_div = np.power(10000.0, 2.0 * even_idx / d, dtype=np.float32)
    odd_div = np.power(10000.0, 2.0 * odd_idx / d, dtype=np.float32)
    pe = np.zeros((max_seq, d), dtype=np.float32)
    pe[:, 0::2] = np.sin(pos / even_div, dtype=np.float32)
    pe[:, 1::2] = np.cos(pos / odd_div, dtype=np.float32)
    return pe


@functools.lru_cache(maxsize=None)
def _make_embed(n_rows, d, interpret=False):
    assert n_rows % (_NW * _MAX_SEQ) == 0
    rows_per_w = n_rows // _NW
    chunk = _MAX_SEQ
    n_chunks = rows_per_w // chunk
    scale = float(math.sqrt(d))

    mesh = plsc.VectorSubcoreMesh(
        core_axis_name="c", subcore_axis_name="s",
        num_cores=_NC, num_subcores=_NS)

    def body(x_hbm, table_hbm, pe_hbm, out_hbm, idx_v, rows_v, pe_v, gsem):
        wid = lax.axis_index("s") * _NC + lax.axis_index("c")
        base = wid * rows_per_w
        pltpu.sync_copy(pe_hbm, pe_v)

        def step(it, carry):
            off = base + it * chunk
            pltpu.sync_copy(x_hbm.at[pl.ds(off, chunk)], idx_v)
            pltpu.async_copy(table_hbm.at[idx_v], rows_v, gsem).wait()

            def row(r, c2):
                for c in range(d // _L):
                    sl = pl.ds(c * _L, _L)
                    rows_v[r, sl] = rows_v[r, sl] * scale + pe_v[r, sl]
                return c2

            lax.fori_loop(0, chunk, row, 0)
            pltpu.sync_copy(rows_v, out_hbm.at[pl.ds(off, chunk)])
            return carry

        lax.fori_loop(0, n_chunks, step, 0)

    return pl.kernel(
        body,
        out_type=jax.ShapeDtypeStruct((n_rows, d), jnp.float32),
        mesh=mesh,
        scratch_types=[
            pltpu.VMEM((chunk,), jnp.int32),
            pltpu.VMEM((chunk, d), jnp.float32),
            pltpu.VMEM((_MAX_SEQ, d), jnp.float32),
            pltpu.SemaphoreType.DMA,
        ],
        compiler_params=pltpu.CompilerParams(use_tc_tiling_on_sc=False),
        interpret=interpret,
    )


def kernel(x, table):
    b, s = x.shape
    d = table.shape[1]
    x_flat = x.reshape(-1).astype(jnp.int32)
    pe = jnp.asarray(_pe_np(_MAX_SEQ, d))
    out = _make_embed(b * s, d)(x_flat, table, pe)
    return out.reshape(b, s, d)


# speedup vs baseline: 1.2153x; 1.2153x over previous
import functools
import math

import numpy as np
import jax
import jax.numpy as jnp
from jax import lax
from jax.experimental import pallas as pl
from jax.experimental.pallas import tpu as pltpu
from jax.experimental.pallas import tpu_sc as plsc

_MAX_SEQ = 200
_D = 64
_L = 16
_NC, _NS = 2, 16
_NW = _NC * _NS
_NBUF = 2


def _pe_np(max_seq, d):
    pos = np.arange(max_seq, dtype=np.float32)[:, None]
    even_idx = np.arange(0, d, 2, dtype=np.float32)
    odd_idx = np.arange(1, d, 2, dtype=np.float32)
    even_div = np.power(10000.0, 2.0 * even_idx / d, dtype=np.float32)
    odd_div = np.power(10000.0, 2.0 * odd_idx / d, dtype=np.float32)
    pe = np.zeros((max_seq, d), dtype=np.float32)
    pe[:, 0::2] = np.sin(pos / even_div, dtype=np.float32)
    pe[:, 1::2] = np.cos(pos / odd_div, dtype=np.float32)
    return pe


@functools.lru_cache(maxsize=None)
def _make_embed(n_rows, d):
    assert n_rows % (_NW * _MAX_SEQ) == 0
    rows_per_w = n_rows // _NW
    chunk = _MAX_SEQ
    n_chunks = rows_per_w // chunk
    assert n_chunks % _NBUF == 0
    scale = float(math.sqrt(d))

    mesh = plsc.VectorSubcoreMesh(
        core_axis_name="c", subcore_axis_name="s",
        num_cores=_NC, num_subcores=_NS)

    def body(x_hbm, table_hbm, pe_hbm, out_hbm,
             idx_v, pe_v, gbuf, obuf, gsems, osems):
        wid = lax.axis_index("s") * _NC + lax.axis_index("c")
        base = wid * rows_per_w
        pltpu.sync_copy(x_hbm.at[pl.ds(base, rows_per_w)], idx_v)
        pltpu.sync_copy(pe_hbm, pe_v)

        def idx_slice(it):
            off = pl.multiple_of(it * chunk, 8)
            return idx_v.at[pl.ds(off, chunk)]

        def start_gather(it, b):
            pltpu.async_copy(table_hbm.at[idx_slice(it)], gbuf[b], gsems[b])

        def wait_gather(it, b):
            pltpu.make_async_copy(
                table_hbm.at[idx_slice(it)], gbuf[b], gsems[b]).wait()

        def start_store(it, b):
            off = base + it * chunk
            pltpu.async_copy(obuf[b], out_hbm.at[pl.ds(off, chunk)], osems[b])

        def wait_store(it, b):
            off = base + it * chunk
            pltpu.make_async_copy(
                obuf[b], out_hbm.at[pl.ds(off, chunk)], osems[b]).wait()

        for b in range(_NBUF):
            start_gather(b, b)

        def outer(i, carry):
            for b in range(_NBUF):
                it = i * _NBUF + b
                wait_gather(it, b)

                @pl.when(it >= _NBUF)
                def _():
                    wait_store(it - _NBUF, b)

                def fma_row(r, c2):
                    for c in range(d // _L):
                        sl = pl.ds(c * _L, _L)
                        obuf[b][r, sl] = gbuf[b][r, sl] * scale + pe_v[r, sl]
                    return c2

                lax.fori_loop(0, chunk, fma_row, 0)

                @pl.when(it + _NBUF < n_chunks)
                def _():
                    start_gather(it + _NBUF, b)

                start_store(it, b)
            return carry

        lax.fori_loop(0, n_chunks // _NBUF, outer, 0)

        for b in range(_NBUF):
            wait_store(n_chunks - _NBUF + b, b)

    return pl.kernel(
        body,
        out_type=jax.ShapeDtypeStruct((n_rows, d), jnp.float32),
        mesh=mesh,
        scratch_types=[
            pltpu.VMEM((rows_per_w,), jnp.int32),
            pltpu.VMEM((_MAX_SEQ, d), jnp.float32),
            [pltpu.VMEM((chunk, d), jnp.float32) for _ in range(_NBUF)],
            [pltpu.VMEM((chunk, d), jnp.float32) for _ in range(_NBUF)],
            [pltpu.SemaphoreType.DMA for _ in range(_NBUF)],
            [pltpu.SemaphoreType.DMA for _ in range(_NBUF)],
        ],
        compiler_params=pltpu.CompilerParams(use_tc_tiling_on_sc=False),
    )


def kernel(x, table):
    b, s = x.shape
    d = table.shape[1]
    x_flat = x.reshape(-1).astype(jnp.int32)
    pe = jnp.asarray(_pe_np(_MAX_SEQ, d))
    out = _make_embed(b * s, d)(x_flat, table, pe)
    return out.reshape(b, s, d)


# speedup vs baseline: 1.2165x; 1.0010x over previous
import functools
import math

import numpy as np
import jax
import jax.numpy as jnp
from jax import lax
from jax.experimental import pallas as pl
from jax.experimental.pallas import tpu as pltpu
from jax.experimental.pallas import tpu_sc as plsc

_MAX_SEQ = 200
_D = 64
_L = 16
_NC, _NS = 2, 16
_NW = _NC * _NS
_NBUF = 2
_UNROLL = 8


def _pe_np(max_seq, d):
    pos = np.arange(max_seq, dtype=np.float32)[:, None]
    even_idx = np.arange(0, d, 2, dtype=np.float32)
    odd_idx = np.arange(1, d, 2, dtype=np.float32)
    even_div = np.power(10000.0, 2.0 * even_idx / d, dtype=np.float32)
    odd_div = np.power(10000.0, 2.0 * odd_idx / d, dtype=np.float32)
    pe = np.zeros((max_seq, d), dtype=np.float32)
    pe[:, 0::2] = np.sin(pos / even_div, dtype=np.float32)
    pe[:, 1::2] = np.cos(pos / odd_div, dtype=np.float32)
    return pe


@functools.lru_cache(maxsize=None)
def _make_embed(n_rows, d):
    assert n_rows % (_NW * _MAX_SEQ) == 0
    rows_per_w = n_rows // _NW
    chunk = _MAX_SEQ
    n_chunks = rows_per_w // chunk
    assert n_chunks % _NBUF == 0
    scale = float(math.sqrt(d))

    mesh = plsc.VectorSubcoreMesh(
        core_axis_name="c", subcore_axis_name="s",
        num_cores=_NC, num_subcores=_NS)

    def body(x_hbm, table_hbm, pe_hbm, out_hbm,
             idx_v, pe_v, gbuf, obuf, gsems, osems):
        wid = lax.axis_index("s") * _NC + lax.axis_index("c")
        base = wid * rows_per_w
        pltpu.sync_copy(x_hbm.at[pl.ds(base, rows_per_w)], idx_v)
        pltpu.sync_copy(pe_hbm, pe_v)

        def idx_slice(it):
            off = pl.multiple_of(it * chunk, 8)
            return idx_v.at[pl.ds(off, chunk)]

        def start_gather(it, b):
            pltpu.async_copy(table_hbm.at[idx_slice(it)], gbuf[b], gsems[b])

        def wait_gather(it, b):
            pltpu.make_async_copy(
                table_hbm.at[idx_slice(it)], gbuf[b], gsems[b]).wait()

        def start_store(it, b):
            off = base + it * chunk
            pltpu.async_copy(obuf[b], out_hbm.at[pl.ds(off, chunk)], osems[b])

        def wait_store(it, b):
            off = base + it * chunk
            pltpu.make_async_copy(
                obuf[b], out_hbm.at[pl.ds(off, chunk)], osems[b]).wait()

        for b in range(_NBUF):
            start_gather(b, b)

        def outer(i, carry):
            for b in range(_NBUF):
                it = i * _NBUF + b
                wait_gather(it, b)

                @pl.when(it >= _NBUF)
                def _():
                    wait_store(it - _NBUF, b)

                def fma_rows(rr, c2):
                    r0 = rr * _UNROLL
                    for u in range(_UNROLL):
                        r = r0 + u
                        for c in range(d // _L):
                            sl = pl.ds(c * _L, _L)
                            obuf[b][r, sl] = (
                                gbuf[b][r, sl] * scale + pe_v[r, sl])
                    return c2

                lax.fori_loop(0, chunk // _UNROLL, fma_rows, 0)

                @pl.when(it + _NBUF < n_chunks)
                def _():
                    start_gather(it + _NBUF, b)

                start_store(it, b)
            return carry

        lax.fori_loop(0, n_chunks // _NBUF, outer, 0)

        for b in range(_NBUF):
            wait_store(n_chunks - _NBUF + b, b)

    return pl.kernel(
        body,
        out_type=jax.ShapeDtypeStruct((n_rows, d), jnp.float32),
        mesh=mesh,
        scratch_types=[
            pltpu.VMEM((rows_per_w,), jnp.int32),
            pltpu.VMEM((_MAX_SEQ, d), jnp.float32),
            [pltpu.VMEM((chunk, d), jnp.float32) for _ in range(_NBUF)],
            [pltpu.VMEM((chunk, d), jnp.float32) for _ in range(_NBUF)],
            [pltpu.SemaphoreType.DMA for _ in range(_NBUF)],
            [pltpu.SemaphoreType.DMA for _ in range(_NBUF)],
        ],
        compiler_params=pltpu.CompilerParams(use_tc_tiling_on_sc=False),
    )


def kernel(x, table):
    b, s = x.shape
    d = table.shape[1]
    x_flat = x.reshape(-1).astype(jnp.int32)
    pe = jnp.asarray(_pe_np(_MAX_SEQ, d))
    out = _make_embed(b * s, d)(x_flat, table, pe)
    return out.reshape(b, s, d)
